# Initial kernel scaffold; baseline (speedup 1.0000x reference)
import jax
import jax.numpy as jnp
from jax.experimental import pallas as pl


def kernel(logits, y):
    raise NotImplementedError("write your pallas kernel here")


# speedup vs baseline: 1.6621x; 1.6621x over previous
import jax
import jax.numpy as jnp
from jax.experimental import pallas as pl

_ROWS, _COLS = 1024, 100000
_BLOCK_ROWS = 8
_GRID = _ROWS // _BLOCK_ROWS


def _cw_kernel(y_ref, x_ref, out_ref):
    x = x_ref[...]
    yb = y_ref[0, 0, :]
    cols = jax.lax.broadcasted_iota(jnp.int32, x.shape, 1)
    is_y = cols == yb[:, None]
    neg = jnp.float32(-jnp.inf)
    best_other = jnp.max(jnp.where(is_y, neg, x), axis=-1)
    correct = jnp.max(jnp.where(is_y, x, neg), axis=-1)
    out_ref[0, 0, :] = jnp.maximum(correct - best_other, 0.0)


@jax.jit
def kernel(logits, y):
    y3 = y.astype(jnp.int32).reshape(_GRID, 1, _BLOCK_ROWS)
    out = pl.pallas_call(
        _cw_kernel,
        grid=(_GRID,),
        in_specs=[
            pl.BlockSpec((1, 1, _BLOCK_ROWS), lambda i: (i, 0, 0)),
            pl.BlockSpec((_BLOCK_ROWS, _COLS), lambda i: (i, 0)),
        ],
        out_specs=pl.BlockSpec((1, 1, _BLOCK_ROWS), lambda i: (i, 0, 0)),
        out_shape=jax.ShapeDtypeStruct((_GRID, 1, _BLOCK_ROWS), logits.dtype),
    )(y3, logits)
    return out.reshape(_ROWS)
